# baseline (speedup 1.0000x reference)
import jax
import jax.numpy as jnp
from jax.experimental import pallas as pl
import jax.experimental.pallas.tpu as pltpu

KH = 3
KW = 3
RH = 32
FHALO = 512


def _conv_kernel(w_ref, m_ref, xm_ref, xh_ref, o_ref):
    f, rh, wo = o_ref.shape
    c = xm_ref.shape[0]
    n = xm_ref.shape[1]
    w = n // rh
    w_eff = (w_ref[...] * m_ref[...]).astype(jnp.bfloat16)
    flat = jnp.concatenate(
        [xm_ref[...], xh_ref[...]], axis=1
    ).astype(jnp.bfloat16)
    pieces = []
    for i in range(KH):
        for j in range(KW):
            off = w * i + j
            rolled = flat if off == 0 else jnp.roll(flat, -off, axis=1)
            pieces.append(rolled[:, :n])
    xcol = jnp.concatenate(pieces, axis=0)
    m = jax.lax.dot_general(
        w_eff, xcol, (((1,), (0,)), ((), ())),
        preferred_element_type=jnp.float32,
    ).reshape(f, rh, w)
    o_ref[...] = m[:, :, :wo]


def kernel(x, kernel_values, kernel_mask):
    b, c, h, w = x.shape
    f = kernel_values.shape[0]
    ho = h - KH + 1
    wo = w - KW + 1
    hw = h * w
    fb = RH * w
    n_rb = hw // fb
    n_hb = hw // FHALO
    ratio = fb // FHALO
    pd = c * KH * KW

    xflat = x.reshape(b, c, hw)

    out = pl.pallas_call(
        _conv_kernel,
        grid=(b, n_rb),
        in_specs=[
            pl.BlockSpec((f, pd), lambda bi, ri: (0, 0)),
            pl.BlockSpec((f, pd), lambda bi, ri: (0, 0)),
            pl.BlockSpec((pl.squeezed, c, fb), lambda bi, ri: (bi, 0, ri)),
            pl.BlockSpec(
                (pl.squeezed, c, FHALO),
                lambda bi, ri: (bi, 0, jnp.minimum(ratio * ri + ratio, n_hb - 1)),
            ),
        ],
        out_specs=pl.BlockSpec(
            (pl.squeezed, f, RH, wo), lambda bi, ri: (bi, 0, ri, 0)
        ),
        out_shape=jax.ShapeDtypeStruct((b, f, ho, wo), jnp.float32),
        compiler_params=pltpu.CompilerParams(
            dimension_semantics=("parallel", "arbitrary"),
        ),
    )(kernel_values, kernel_mask, xflat, xflat)
    return out


# speedup vs baseline: 1.0079x; 1.0079x over previous
import jax
import jax.numpy as jnp
from jax.experimental import pallas as pl
import jax.experimental.pallas.tpu as pltpu

KH = 3
KW = 3
RH = 32
FHALO = 512


def _conv_kernel(w_ref, m_ref, xm_ref, o_ref):
    f, rh, wo = o_ref.shape
    c = xm_ref.shape[1]
    n = xm_ref.shape[2] - FHALO
    w = n // rh
    w_eff = (w_ref[...] * m_ref[...]).astype(jnp.bfloat16)
    flat = xm_ref[0].astype(jnp.bfloat16)
    pieces = []
    for i in range(KH):
        for j in range(KW):
            off = w * i + j
            rolled = flat if off == 0 else jnp.roll(flat, -off, axis=1)
            pieces.append(rolled[:, :n])
    xcol = jnp.concatenate(pieces, axis=0)
    m = jax.lax.dot_general(
        w_eff, xcol, (((1,), (0,)), ((), ())),
        preferred_element_type=jnp.float32,
    ).reshape(f, rh, w)
    o_ref[...] = m[:, :, :wo]


def kernel(x, kernel_values, kernel_mask):
    b, c, h, w = x.shape
    f = kernel_values.shape[0]
    ho = h - KH + 1
    wo = w - KW + 1
    hw = h * w
    fb = RH * w
    n_rb = hw // fb
    n_hb = hw // FHALO
    ratio = fb // FHALO
    pd = c * KH * KW

    xflat = x.reshape(b, c, hw)

    out = pl.pallas_call(
        _conv_kernel,
        grid=(b, n_rb),
        in_specs=[
            pl.BlockSpec((f, pd), lambda bi, ri: (0, 0)),
            pl.BlockSpec((f, pd), lambda bi, ri: (0, 0)),
            pl.BlockSpec(
                (pl.Element(1), pl.Element(c), pl.Element(fb + FHALO, (0, FHALO))),
                lambda bi, ri: (bi, 0, ri * fb),
            ),
        ],
        out_specs=pl.BlockSpec(
            (pl.squeezed, f, RH, wo), lambda bi, ri: (bi, 0, ri, 0)
        ),
        out_shape=jax.ShapeDtypeStruct((b, f, ho, wo), jnp.float32),
        compiler_params=pltpu.CompilerParams(
            dimension_semantics=("parallel", "arbitrary"),
        ),
    )(kernel_values, kernel_mask, xflat)
    return out


# speedup vs baseline: 1.6777x; 1.6646x over previous
import jax
import jax.numpy as jnp
from jax.experimental import pallas as pl
import jax.experimental.pallas.tpu as pltpu

KH = 3
KW = 3
RH = 32
HALO = 8


def _conv_kernel(w_ref, m_ref, xm_ref, xh_ref, o_ref):
    f, rh, wo = o_ref.shape
    c, _, w = xm_ref.shape
    w_eff = (w_ref[...] * m_ref[...]).astype(jnp.bfloat16)
    xfull = jnp.concatenate(
        [xm_ref[...], xh_ref[...]], axis=1
    ).astype(jnp.bfloat16)
    wp = 256
    xpad = jnp.concatenate(
        [xfull[:, :rh + KH - 1, :],
         jnp.zeros((c, rh + KH - 1, wp - w), jnp.bfloat16)],
        axis=2,
    )
    flat = xpad.reshape(c, (rh + KH - 1) * wp)
    n = rh * wp
    pieces = []
    for j in range(KW):
        rolled = flat if j == 0 else jnp.roll(flat, -j, axis=1)
        for i in range(KH):
            pieces.append((rolled[:, wp * i:wp * i + n], i, j))
    pieces.sort(key=lambda t: (t[1], t[2]))
    xcol = jnp.concatenate([p for p, _, _ in pieces], axis=0)
    m = jax.lax.dot_general(
        w_eff, xcol, (((1,), (0,)), ((), ())),
        preferred_element_type=jnp.float32,
    ).reshape(f, rh, wp)
    o_ref[...] = m[:, :, :wo]


def kernel(x, kernel_values, kernel_mask):
    b, c, h, w = x.shape
    f = kernel_values.shape[0]
    ho = h - KH + 1
    wo = w - KW + 1
    n_rb = h // RH
    n_hb = h // HALO
    pd = c * KH * KW
    ratio = RH // HALO

    out = pl.pallas_call(
        _conv_kernel,
        grid=(b, n_rb),
        in_specs=[
            pl.BlockSpec((f, pd), lambda bi, ri: (0, 0)),
            pl.BlockSpec((f, pd), lambda bi, ri: (0, 0)),
            pl.BlockSpec((pl.squeezed, c, RH, w), lambda bi, ri: (bi, 0, ri, 0)),
            pl.BlockSpec(
                (pl.squeezed, c, HALO, w),
                lambda bi, ri: (bi, 0, jnp.minimum(ratio * ri + ratio, n_hb - 1), 0),
            ),
        ],
        out_specs=pl.BlockSpec(
            (pl.squeezed, f, RH, wo), lambda bi, ri: (bi, 0, ri, 0)
        ),
        out_shape=jax.ShapeDtypeStruct((b, f, ho, wo), jnp.float32),
        compiler_params=pltpu.CompilerParams(
            dimension_semantics=("parallel", "arbitrary"),
        ),
    )(kernel_values, kernel_mask, x, x)
    return out


# speedup vs baseline: 1.7167x; 1.0232x over previous
import jax
import jax.numpy as jnp
from jax.experimental import pallas as pl
import jax.experimental.pallas.tpu as pltpu

KH = 3
KW = 3
RH = 56
HALO = 8


def _conv_kernel(w_ref, m_ref, xm_ref, xh_ref, o_ref):
    f, rh, wo = o_ref.shape
    c, _, w = xm_ref.shape
    w_eff = (w_ref[...] * m_ref[...]).astype(jnp.bfloat16)
    xfull = jnp.concatenate(
        [xm_ref[...], xh_ref[...]], axis=1
    ).astype(jnp.bfloat16)
    wp = 256
    xpad = jnp.concatenate(
        [xfull[:, :rh + KH - 1, :],
         jnp.zeros((c, rh + KH - 1, wp - w), jnp.bfloat16)],
        axis=2,
    )
    flat = xpad.reshape(c, (rh + KH - 1) * wp)
    n = rh * wp
    pieces = []
    for j in range(KW):
        rolled = flat if j == 0 else jnp.roll(flat, -j, axis=1)
        for i in range(KH):
            pieces.append((rolled[:, wp * i:wp * i + n], i, j))
    pieces.sort(key=lambda t: (t[1], t[2]))
    xcol = jnp.concatenate([p for p, _, _ in pieces], axis=0)
    m = jax.lax.dot_general(
        w_eff, xcol, (((1,), (0,)), ((), ())),
        preferred_element_type=jnp.float32,
    ).reshape(f, rh, wp)
    o_ref[...] = m[:, :, :wo]


def kernel(x, kernel_values, kernel_mask):
    b, c, h, w = x.shape
    f = kernel_values.shape[0]
    ho = h - KH + 1
    wo = w - KW + 1
    n_rb = h // RH
    n_hb = h // HALO
    pd = c * KH * KW
    ratio = RH // HALO

    out = pl.pallas_call(
        _conv_kernel,
        grid=(b, n_rb),
        in_specs=[
            pl.BlockSpec((f, pd), lambda bi, ri: (0, 0)),
            pl.BlockSpec((f, pd), lambda bi, ri: (0, 0)),
            pl.BlockSpec((pl.squeezed, c, RH, w), lambda bi, ri: (bi, 0, ri, 0)),
            pl.BlockSpec(
                (pl.squeezed, c, HALO, w),
                lambda bi, ri: (bi, 0, jnp.minimum(ratio * ri + ratio, n_hb - 1), 0),
            ),
        ],
        out_specs=pl.BlockSpec(
            (pl.squeezed, f, RH, wo), lambda bi, ri: (bi, 0, ri, 0)
        ),
        out_shape=jax.ShapeDtypeStruct((b, f, ho, wo), jnp.float32),
        compiler_params=pltpu.CompilerParams(
            dimension_semantics=("parallel", "arbitrary"),
        ),
    )(kernel_values, kernel_mask, x, x)
    return out
